# baseline (speedup 1.0000x reference)
import functools

import jax
import jax.numpy as jnp
from jax import lax
from jax.experimental import pallas as pl
from jax.experimental.pallas import tpu as pltpu
from jax.experimental.pallas import tpu_sc as plsc

_B = 16384
_C = 1000
_N = 1000
_NP = 1024
_NC = 2
_NS = 16
_NW = _NC * _NS
_BPW = _B // _NW
_R = 16
_NCH = _BPW // _R

_mesh = plsc.VectorSubcoreMesh(core_axis_name="c", subcore_axis_name="s",
                               num_cores=1)


@functools.partial(
    pl.kernel,
    out_type=jax.ShapeDtypeStruct((_NS, 16), jnp.float32),
    mesh=_mesh,
    scratch_types=[
        pltpu.VMEM((_NCH, _R), jnp.int32),
        pltpu.VMEM((_R, _N), jnp.float32),
        pltpu.VMEM((_R, _N), jnp.float32),
        pltpu.VMEM((_R, _NP), jnp.float32),
        pltpu.VMEM((_R, _NP), jnp.float32),
        pltpu.VMEM((16,), jnp.float32),
        pltpu.SemaphoreType.DMA,
        pltpu.SemaphoreType.DMA,
        pltpu.SemaphoreType.DMA,
        pltpu.SemaphoreType.DMA,
    ],
)
def _sc_mse(x_hbm, y_hbm, ctr_hbm, out_hbm,
            idx_v, xc0, xc1, g0, g1, acc_v, si0, si1, sg0, sg1):
    xc = (xc0, xc1)
    g = (g0, g1)
    sem_in = (si0, si1)
    sem_g = (sg0, sg1)

    wid = lax.axis_index("s")
    base = wid * _BPW
    lane = lax.broadcasted_iota(jnp.int32, (16,), 0)

    def ld_idx(j, carry):
        pltpu.sync_copy(y_hbm.at[pl.ds(base + j * _R, _R)], idx_v.at[j])
        return carry
    lax.fori_loop(0, _NCH, ld_idx, 0)

    acc_v[...] = jnp.zeros((16,), jnp.float32)

    for b in range(2):
        pltpu.async_copy(x_hbm.at[pl.ds(base + b * _R, _R)], xc[b], sem_in[b])
        pltpu.async_copy(ctr_hbm.at[idx_v.at[b]], g[b], sem_g[b])

    def pair(jj, carry):
        for b in range(2):
            j = jj * 2 + b
            pltpu.make_async_copy(
                x_hbm.at[pl.ds(base + j * _R, _R)], xc[b], sem_in[b]).wait()
            pltpu.make_async_copy(
                ctr_hbm.at[idx_v.at[j]], g[b], sem_g[b]).wait()

            def row(r, acc_r):
                def col(ci, acc_c):
                    vx = xc[b][r, pl.ds(ci * 16, 16)]
                    vg = g[b][r, pl.ds(ci * 16, 16)]
                    d = vx - vg
                    return acc_c + d * d
                acc_r = lax.fori_loop(0, 62, col, acc_r, unroll=8)
                tx = xc[b][r, pl.ds(_N - 16, 16)]
                tg = g[b][r, pl.ds(_N - 16, 16)]
                d = jnp.where(lane >= 8, tx - tg, 0.0)
                return acc_r + d * d
            acc = lax.fori_loop(0, _R, row, acc_v[...])
            acc_v[...] = acc

            @pl.when(j < _NCH - 2)
            def _prefetch():
                pltpu.async_copy(
                    x_hbm.at[pl.ds(base + (j + 2) * _R, _R)], xc[b],
                    sem_in[b])
                pltpu.async_copy(
                    ctr_hbm.at[idx_v.at[j + 2]], g[b], sem_g[b])
        return carry

    lax.fori_loop(0, _NCH // 2, pair, 0)

    pltpu.sync_copy(acc_v, out_hbm.at[wid])


def kernel(x, y, centers):
    y32 = y.astype(jnp.int32)
    ctr_p = jnp.pad(centers, ((0, 0), (0, _NP - _N)))
    half = _B // 2
    p0 = _sc_mse(x[:half], y32[:half], ctr_p)
    p1 = _sc_mse(x[half:], y32[half:], ctr_p)
    total = jnp.sum(p0) + jnp.sum(p1)
    return (total * (1.0 / (_B * _N))).astype(jnp.float32)


# speedup vs baseline: 1.9996x; 1.9996x over previous
import functools

import jax
import jax.numpy as jnp
from jax import lax
from jax.experimental import pallas as pl
from jax.experimental.pallas import tpu as pltpu
from jax.experimental.pallas import tpu_sc as plsc

_B = 16384
_C = 1000
_N = 1000
_NP = 1024
_NC = 2
_NS = 16
_NW = _NC * _NS

_B_SC = 7168
_B_TC = _B - _B_SC
_BPW = _B_SC // _NW
_R = 16
_NCH = _BPW // _R

_BM = 1024
_TC_OFF = _B_SC // _BM

_mesh = plsc.VectorSubcoreMesh(core_axis_name="c", subcore_axis_name="s")


@functools.partial(
    pl.kernel,
    out_type=jax.ShapeDtypeStruct((_NW, 16), jnp.float32),
    mesh=_mesh,
    scratch_types=[
        pltpu.VMEM((_NCH, _R), jnp.int32),
        pltpu.VMEM((_R, _N), jnp.float32),
        pltpu.VMEM((_R, _N), jnp.float32),
        pltpu.VMEM((_R, _NP), jnp.float32),
        pltpu.VMEM((_R, _NP), jnp.float32),
        pltpu.VMEM((16,), jnp.float32),
        pltpu.SemaphoreType.DMA,
        pltpu.SemaphoreType.DMA,
        pltpu.SemaphoreType.DMA,
        pltpu.SemaphoreType.DMA,
    ],
)
def _sc_mse(x_hbm, y_hbm, ctr_hbm, out_hbm,
            idx_v, xc0, xc1, g0, g1, acc_v, si0, si1, sg0, sg1):
    xc = (xc0, xc1)
    g = (g0, g1)
    sem_in = (si0, si1)
    sem_g = (sg0, sg1)

    c = lax.axis_index("c")
    s = lax.axis_index("s")
    wid = c * _NS + s
    base = wid * _BPW
    lane = lax.broadcasted_iota(jnp.int32, (16,), 0)

    def ld_idx(j, carry):
        pltpu.sync_copy(y_hbm.at[pl.ds(base + j * _R, _R)], idx_v.at[j])
        return carry
    lax.fori_loop(0, _NCH, ld_idx, 0)

    acc_v[...] = jnp.zeros((16,), jnp.float32)

    for b in range(2):
        pltpu.async_copy(x_hbm.at[pl.ds(base + b * _R, _R)], xc[b], sem_in[b])
        pltpu.async_copy(ctr_hbm.at[idx_v.at[b]], g[b], sem_g[b])

    def pair(jj, carry):
        for b in range(2):
            j = jj * 2 + b
            pltpu.make_async_copy(
                x_hbm.at[pl.ds(base + j * _R, _R)], xc[b], sem_in[b]).wait()
            pltpu.make_async_copy(
                ctr_hbm.at[idx_v.at[j]], g[b], sem_g[b]).wait()

            def row(r, acc_r):
                def col(ci, acc_c):
                    vx = xc[b][r, pl.ds(ci * 16, 16)]
                    vg = g[b][r, pl.ds(ci * 16, 16)]
                    d = vx - vg
                    return acc_c + d * d
                acc_r = lax.fori_loop(0, 62, col, acc_r, unroll=8)
                tx = xc[b][r, pl.ds(_N - 16, 16)]
                tg = g[b][r, pl.ds(_N - 16, 16)]
                d = jnp.where(lane >= 8, tx - tg, 0.0)
                return acc_r + d * d
            acc = lax.fori_loop(0, _R, row, acc_v[...])
            acc_v[...] = acc

            @pl.when(j < _NCH - 2)
            def _prefetch():
                pltpu.async_copy(
                    x_hbm.at[pl.ds(base + (j + 2) * _R, _R)], xc[b],
                    sem_in[b])
                pltpu.async_copy(
                    ctr_hbm.at[idx_v.at[j + 2]], g[b], sem_g[b])
        return carry

    lax.fori_loop(0, _NCH // 2, pair, 0)

    pltpu.sync_copy(acc_v, out_hbm.at[wid])


def _tc_mse_kernel(x_ref, y_ref, centers_ref, out_ref):
    i = pl.program_id(0)

    y_blk = y_ref[...]
    classes = lax.broadcasted_iota(jnp.int32, (_BM, _C), 1)
    onehot = (classes == y_blk).astype(jnp.float32)
    gathered = jnp.dot(onehot, centers_ref[...],
                       preferred_element_type=jnp.float32)
    d = x_ref[...] - gathered
    part = jnp.sum(d * d)

    @pl.when(i == 0)
    def _init():
        out_ref[0, 0] = 0.0

    out_ref[0, 0] += part


def kernel(x, y, centers):
    y32 = y.astype(jnp.int32)
    ctr_p = jnp.pad(centers, ((0, 0), (0, _NP - _N)))
    sc_partials = _sc_mse(x, y32, ctr_p)

    y2d = y32.reshape(_B, 1)
    tc_total = pl.pallas_call(
        _tc_mse_kernel,
        grid=(_B_TC // _BM,),
        in_specs=[
            pl.BlockSpec((_BM, _N), lambda i: (_TC_OFF + i, 0)),
            pl.BlockSpec((_BM, 1), lambda i: (_TC_OFF + i, 0)),
            pl.BlockSpec((_C, _N), lambda i: (0, 0)),
        ],
        out_specs=pl.BlockSpec((1, 1), lambda i: (0, 0),
                               memory_space=pltpu.SMEM),
        out_shape=jax.ShapeDtypeStruct((1, 1), jnp.float32),
    )(x, y2d, centers)

    total = jnp.sum(sc_partials) + tc_total[0, 0]
    return (total * (1.0 / (_B * _N))).astype(jnp.float32)
